# baseline (speedup 1.0000x reference)
import functools

import jax
import jax.numpy as jnp
from jax.experimental import pallas as pl
from jax.experimental.pallas import tpu as pltpu

N_TOKENS = 32768
N_EMBD = 4096
N_CHOICES = 64
HIDDEN = N_EMBD // 2

BT = 256


def _router_body(k_ref, x_ref, w1_ref, b1_ref, w2_ref, b2_ref, o_ref):
    h = jnp.dot(x_ref[...], w1_ref[...], preferred_element_type=jnp.float32)
    h = jnp.maximum(h + b1_ref[...], 0.0)
    logits = jnp.dot(h, w2_ref[...], preferred_element_type=jnp.float32)
    logits = logits + b2_ref[...]

    col = jax.lax.broadcasted_iota(jnp.int32, logits.shape, 1)
    b = jax.lax.bitcast_convert_type(logits, jnp.int32)
    key = b ^ ((b >> 31) & jnp.int32(0x7FFFFFFF))
    key = (key & jnp.int32(~63)) | (jnp.int32(63) - col)
    k1 = jnp.max(key, axis=-1, keepdims=True)
    k2 = jnp.max(
        jnp.where(key == k1, jnp.int32(-0x80000000), key), axis=-1, keepdims=True
    )
    i1 = jnp.int32(63) - (k1 & jnp.int32(63))
    i2 = jnp.int32(63) - (k2 & jnp.int32(63))

    def _unkey(kk):
        ub = kk & jnp.int32(~63)
        return jax.lax.bitcast_convert_type(
            ub ^ ((ub >> 31) & jnp.int32(0x7FFFFFFF)), jnp.float32
        )

    p1 = jax.nn.sigmoid(_unkey(k1) - _unkey(k2))
    k_is_1 = k_ref[0] == 1
    v1 = jnp.where(k_is_1, jnp.float32(1.0), p1)
    v2 = jnp.where(k_is_1, jnp.float32(0.0), 1.0 - p1)
    o_ref[...] = jnp.where(col == i1, v1, jnp.where(col == i2, v2, 0.0))


@functools.partial(jax.jit, static_argnames=("interpret",))
def _router(x, W1, b1, W2, b2, k, interpret=False):
    grid = (N_TOKENS // BT,)
    return pl.pallas_call(
        _router_body,
        grid=grid,
        in_specs=[
            pl.BlockSpec(memory_space=pltpu.SMEM),
            pl.BlockSpec((BT, N_EMBD), lambda i: (i, 0)),
            pl.BlockSpec((N_EMBD, HIDDEN), lambda i: (0, 0)),
            pl.BlockSpec((1, HIDDEN), lambda i: (0, 0)),
            pl.BlockSpec((HIDDEN, N_CHOICES), lambda i: (0, 0)),
            pl.BlockSpec((1, N_CHOICES), lambda i: (0, 0)),
        ],
        out_specs=pl.BlockSpec((BT, N_CHOICES), lambda i: (i, 0)),
        out_shape=jax.ShapeDtypeStruct((N_TOKENS, N_CHOICES), jnp.float32),
        compiler_params=pltpu.CompilerParams(vmem_limit_bytes=100 * 1024 * 1024),
        interpret=interpret,
    )(k, x, W1, b1, W2, b2)


def kernel(x, W1, b1, W2, b2, k, training):
    k_arr = jnp.asarray(k, jnp.int32).reshape((1,))
    return _router(
        x, W1, b1.reshape(1, HIDDEN), W2, b2.reshape(1, N_CHOICES), k_arr
    )
